# baseline (speedup 1.0000x reference)
import functools

import jax
import jax.numpy as jnp
from jax import lax
from jax.experimental import pallas as pl
from jax.experimental.pallas import tpu as pltpu
from jax.experimental.pallas import tpu_sc as plsc

_PARTNER = {0: 1, 1: 3, 2: 4, 5: 7, 6: 8, 7: 9, 8: 10, 11: 13, 12: 14,
            13: 15, 14: 16}

_P = 4096 * 128
_NPLANES = 51
_N = _NPLANES * _P
_NW = 32
_SS = _P // _NW
_NV = _SS // 16
_R = 7
_L = 5
_ZS = _SS // 2

_DELTA = {}
_INACTIVE = []
for _j in range(17):
    for _k in range(3):
        _w = 3 * _j + _k
        if _j in _PARTNER:
            _DELTA[_w] = 3 * _PARTNER[_j] + _k
        else:
            _INACTIVE.append(_w)

_ACTIVE_J = sorted(_PARTNER)


@functools.partial(
    pl.kernel,
    out_type=jax.ShapeDtypeStruct((_N,), jnp.float32),
    mesh=plsc.VectorSubcoreMesh(core_axis_name="c", subcore_axis_name="s"),
    compiler_params=pltpu.CompilerParams(needs_layout_passes=False),
    scratch_types=(
        [pltpu.VMEM((_SS,), jnp.float32) for _ in range(_R)]
        + [pltpu.VMEM((_ZS,), jnp.float32)]
        + [pltpu.SemaphoreType.DMA for _ in range(2 * _R + 1)]
    ),
)
def _sc_joint2bone(x_hbm, out_hbm, *refs):
    ring = refs[:_R]
    zv = refs[_R]
    lsem = refs[_R + 1:2 * _R + 1]
    ssem = refs[2 * _R + 1:3 * _R + 1]
    zsem = refs[3 * _R + 1]

    wid = lax.axis_index("s") * 2 + lax.axis_index("c")
    woff = wid * _SS

    zvec = jnp.zeros((16,), jnp.float32)

    @plsc.parallel_loop(0, _ZS // 16, 1, unroll=8)
    def zero_body(i):
        zv[pl.ds(i * 16, 16)] = zvec

    koffs = [lax.rem(wid + c, 3) * _P + woff for c in range(3)]

    def hbm_off(g):
        chain, i = divmod(g, 17)
        return koffs[chain] + 3 * i * _P

    def start_load(g):
        pltpu.async_copy(x_hbm.at[pl.ds(hbm_off(g), _SS)], ring[g % _R],
                         lsem[g % _R])

    def wait_load(g):
        pltpu.make_async_copy(x_hbm.at[pl.ds(0, _SS)], ring[g % _R],
                              lsem[g % _R]).wait()

    def start_store(g):
        pltpu.async_copy(ring[g % _R], out_hbm.at[pl.ds(hbm_off(g), _SS)],
                         ssem[g % _R])

    def wait_store(g):
        pltpu.make_async_copy(ring[g % _R], out_hbm.at[pl.ds(0, _SS)],
                              ssem[g % _R]).wait()

    loads_waited = set()
    stores_unwaited = set()
    for g in range(_L):
        start_load(g)
    for g in range(_NPLANES):
        chain, i = divmod(g, 17)
        nxt = g + _L
        if nxt < _NPLANES:
            prev = nxt - _R
            if prev in stores_unwaited:
                wait_store(prev)
                stores_unwaited.discard(prev)
            start_load(nxt)
        if i in _PARTNER:
            pg = chain * 17 + _PARTNER[i]
            for q in (g, pg):
                if q not in loads_waited:
                    wait_load(q)
                    loads_waited.add(q)
            xa = ring[g % _R]
            xb = ring[pg % _R]

            @plsc.parallel_loop(0, _NV, 1, unroll=8)
            def sub_body(ii, xa=xa, xb=xb):
                s = pl.ds(ii * 16, 16)
                xa[s] = xa[s] - xb[s]

            start_store(g)
            stores_unwaited.add(g)
        else:
            for h in range(2):
                pltpu.async_copy(
                    zv, out_hbm.at[pl.ds(hbm_off(g) + h * _ZS, _ZS)], zsem)
    for g in sorted(stores_unwaited):
        wait_store(g)
    for _ in range(2 * len(_INACTIVE)):
        pltpu.make_async_copy(zv, out_hbm.at[pl.ds(0, _ZS)], zsem).wait()


@jax.jit
def kernel(joint):
    x = jnp.transpose(joint, (2, 3, 0, 1)).reshape(-1)
    out = _sc_joint2bone(x)
    return jnp.transpose(out.reshape(17, 3, 4096, 128), (2, 3, 0, 1))


# speedup vs baseline: 1.0037x; 1.0037x over previous
import functools

import jax
import jax.numpy as jnp
from jax import lax
from jax.experimental import pallas as pl
from jax.experimental.pallas import tpu as pltpu
from jax.experimental.pallas import tpu_sc as plsc

_PARTNER = {0: 1, 1: 3, 2: 4, 5: 7, 6: 8, 7: 9, 8: 10, 11: 13, 12: 14,
            13: 15, 14: 16}

_P = 4096 * 128
_NPLANES = 51
_N = _NPLANES * _P
_NW = 32
_SS = _P // _NW
_NV = _SS // 16
_R = 7
_L = 5
_ZS = _SS // 2

_DELTA = {}
_INACTIVE = []
for _j in range(17):
    for _k in range(3):
        _w = 3 * _j + _k
        if _j in _PARTNER:
            _DELTA[_w] = 3 * _PARTNER[_j] + _k
        else:
            _INACTIVE.append(_w)

_P_ORDER = [3 * _i + _k for _k in range(3) for _i in range(17)]
_POS = {_w: _g for _g, _w in enumerate(_P_ORDER)}


@functools.partial(
    pl.kernel,
    out_type=jax.ShapeDtypeStruct((_N,), jnp.float32),
    mesh=plsc.VectorSubcoreMesh(core_axis_name="c", subcore_axis_name="s"),
    compiler_params=pltpu.CompilerParams(needs_layout_passes=False),
    scratch_types=(
        [pltpu.VMEM((_SS,), jnp.float32) for _ in range(_R)]
        + [pltpu.VMEM((_ZS,), jnp.float32)]
        + [pltpu.SemaphoreType.DMA for _ in range(2 * _R + 1)]
    ),
)
def _sc_joint2bone(x_hbm, out_hbm, *refs):
    ring = refs[:_R]
    zv = refs[_R]
    lsem = refs[_R + 1:2 * _R + 1]
    ssem = refs[2 * _R + 1:3 * _R + 1]
    zsem = refs[3 * _R + 1]

    wid = lax.axis_index("s") * 2 + lax.axis_index("c")
    woff = wid * _SS

    zvec = jnp.zeros((16,), jnp.float32)

    @plsc.parallel_loop(0, _ZS // 16, 1, unroll=8)
    def zero_body(i):
        zv[pl.ds(i * 16, 16)] = zvec

    def slot(w):
        return _POS[w] % _R

    def start_load(w):
        pltpu.async_copy(x_hbm.at[pl.ds(w * _P + woff, _SS)], ring[slot(w)],
                         lsem[slot(w)])

    def wait_load(w):
        pltpu.make_async_copy(x_hbm.at[pl.ds(0, _SS)], ring[slot(w)],
                              lsem[slot(w)]).wait()

    def start_store(w):
        pltpu.async_copy(ring[slot(w)],
                         out_hbm.at[pl.ds(w * _P + woff, _SS)], ssem[slot(w)])

    def wait_store(w):
        pltpu.make_async_copy(ring[slot(w)], out_hbm.at[pl.ds(0, _SS)],
                              ssem[slot(w)]).wait()

    loads_waited = set()
    stores_unwaited = set()
    for g in range(_L):
        start_load(_P_ORDER[g])
    for g in range(_NPLANES):
        w = _P_ORDER[g]
        nxt = g + _L
        if nxt < _NPLANES:
            prev = nxt - _R
            if prev >= 0 and _P_ORDER[prev] in stores_unwaited:
                wait_store(_P_ORDER[prev])
                stores_unwaited.discard(_P_ORDER[prev])
            start_load(_P_ORDER[nxt])
        if w in _DELTA:
            for q in (w, _DELTA[w]):
                if q not in loads_waited:
                    wait_load(q)
                    loads_waited.add(q)
            xa = ring[slot(w)]
            xb = ring[slot(_DELTA[w])]

            @plsc.parallel_loop(0, _NV, 1, unroll=8)
            def sub_body(i, xa=xa, xb=xb):
                s = pl.ds(i * 16, 16)
                xa[s] = xa[s] - xb[s]

            start_store(w)
            stores_unwaited.add(w)
        else:
            for h in range(2):
                pltpu.async_copy(
                    zv, out_hbm.at[pl.ds(w * _P + woff + h * _ZS, _ZS)],
                    zsem)
    for w in [p for p in _P_ORDER if p in stores_unwaited]:
        wait_store(w)
    for _ in range(2 * len(_INACTIVE)):
        pltpu.make_async_copy(zv, out_hbm.at[pl.ds(0, _ZS)], zsem).wait()


@jax.jit
def kernel(joint):
    x = jnp.transpose(joint, (2, 3, 0, 1)).reshape(-1)
    out = _sc_joint2bone(x)
    return jnp.transpose(out.reshape(17, 3, 4096, 128), (2, 3, 0, 1))
